# Initial kernel scaffold; baseline (speedup 1.0000x reference)
import functools

import jax
import jax.numpy as jnp
from jax import lax
from jax.experimental import pallas as pl

_K = 64
_N = 32768
_ROWS = 128
_BLK_R = 8
_CHUNK = 128
_INT32_MIN = jnp.int32(-2147483648)


def _bit_const(bit: int):
    v = 1 << bit
    if v >= 2**31:
        v -= 2**32
    return jnp.int32(v)


def _topk_mask_body(x_ref, o_ref):
    x = x_ref[...]
    r = x.shape[0]
    n = x.shape[1]
    c = n // _CHUNK

    b = lax.bitcast_convert_type(x, jnp.int32)
    sv = b ^ ((b >> 31) & jnp.int32(0x7FFFFFFF))

    p = jnp.zeros((r, 1), jnp.int32)
    for bit in range(31, -1, -1):
        cand = p | _bit_const(bit)
        cand_sv = cand ^ _INT32_MIN
        cnt = jnp.sum((sv >= cand_sv).astype(jnp.int32), axis=1, keepdims=True)
        p = jnp.where(cnt >= _K, cand, p)
    t_sv = p ^ _INT32_MIN

    gt = sv > t_sv
    eq = sv == t_sv
    c_gt = jnp.sum(gt.astype(jnp.int32), axis=1, keepdims=True)
    need = (_K - c_gt).astype(jnp.float32)

    eqf = eq.astype(jnp.float32)
    eq2 = eqf.reshape(r * c, _CHUNK)
    i128 = lax.broadcasted_iota(jnp.int32, (_CHUNK, _CHUNK), 0)
    j128 = lax.broadcasted_iota(jnp.int32, (_CHUNK, _CHUNK), 1)
    tu128 = (i128 < j128).astype(jnp.float32)
    local = jnp.dot(eq2, tu128, preferred_element_type=jnp.float32)
    local = local.reshape(r, c, _CHUNK)

    csum = jnp.sum(eqf.reshape(r, c, _CHUNK), axis=2)
    ic = lax.broadcasted_iota(jnp.int32, (c, c), 0)
    jc = lax.broadcasted_iota(jnp.int32, (c, c), 1)
    tuc = (ic < jc).astype(jnp.float32)
    carry = jnp.dot(csum, tuc, preferred_element_type=jnp.float32)

    prefix = (local + carry[:, :, None]).reshape(r, n)
    keep = gt | (eq & (prefix < need))
    o_ref[...] = jnp.where(keep, x, 0.0)


@jax.jit
def kernel(x):
    grid = (_ROWS // _BLK_R,)
    return pl.pallas_call(
        _topk_mask_body,
        grid=grid,
        in_specs=[pl.BlockSpec((_BLK_R, _N), lambda i: (i, 0))],
        out_specs=pl.BlockSpec((_BLK_R, _N), lambda i: (i, 0)),
        out_shape=jax.ShapeDtypeStruct((_ROWS, _N), jnp.float32),
    )(x)


# speedup vs baseline: 4.4654x; 4.4654x over previous
import jax
import jax.numpy as jnp
import numpy as np
from jax import lax
from jax.experimental import pallas as pl

_K = 64
_N = 32768
_ROWS = 128
_BLK_R = 8
_CHUNK = 128
_INT32_MIN = np.int32(-2147483648)


def _bit_const(bit: int):
    v = 1 << bit
    if v >= 2**31:
        v -= 2**32
    return np.int32(v)


def _topk_mask_body(x_ref, o_ref):
    x = x_ref[...]
    r = x.shape[0]
    n = x.shape[1]
    c = n // _CHUNK

    b = lax.bitcast_convert_type(x, jnp.int32)
    sv = b ^ ((b >> 31) & np.int32(0x7FFFFFFF))

    p = jnp.zeros((r, 1), jnp.int32)
    for bit in range(31, -1, -1):
        cand = p | _bit_const(bit)
        cand_sv = cand ^ _INT32_MIN
        cnt = jnp.sum((sv >= cand_sv).astype(jnp.int32), axis=1, keepdims=True)
        p = jnp.where(cnt >= _K, cand, p)
    t_sv = p ^ _INT32_MIN

    gt = sv > t_sv
    eq = sv == t_sv
    c_gt = jnp.sum(gt.astype(jnp.int32), axis=1, keepdims=True)
    need = (_K - c_gt).astype(jnp.float32)

    eqf = eq.astype(jnp.float32)
    eq2 = eqf.reshape(r * c, _CHUNK)
    i128 = lax.broadcasted_iota(jnp.int32, (_CHUNK, _CHUNK), 0)
    j128 = lax.broadcasted_iota(jnp.int32, (_CHUNK, _CHUNK), 1)
    tu128 = (i128 < j128).astype(jnp.float32)
    local = jnp.dot(eq2, tu128, preferred_element_type=jnp.float32)
    local = local.reshape(r, c, _CHUNK)

    csum = jnp.sum(eqf.reshape(r, c, _CHUNK), axis=2)
    ic = lax.broadcasted_iota(jnp.int32, (c, c), 0)
    jc = lax.broadcasted_iota(jnp.int32, (c, c), 1)
    tuc = (ic < jc).astype(jnp.float32)
    carry = jnp.dot(csum, tuc, preferred_element_type=jnp.float32)

    prefix = (local + carry[:, :, None]).reshape(r, n)
    keep = gt | (eq & (prefix < need))
    o_ref[...] = jnp.where(keep, x, 0.0)


@jax.jit
def kernel(x):
    grid = (_ROWS // _BLK_R,)
    return pl.pallas_call(
        _topk_mask_body,
        grid=grid,
        in_specs=[pl.BlockSpec((_BLK_R, _N), lambda i: (i, 0))],
        out_specs=pl.BlockSpec((_BLK_R, _N), lambda i: (i, 0)),
        out_shape=jax.ShapeDtypeStruct((_ROWS, _N), jnp.float32),
    )(x)
